# baseline (speedup 1.0000x reference)
import functools

import jax
import jax.numpy as jnp
from jax import lax
from jax.experimental import pallas as pl
from jax.experimental.pallas import tpu as pltpu
from jax.experimental.pallas import tpu_sc as plsc

NC = 2
NS = 16
K = 128


def _round_up(a, b):
    return -(-a // b) * b


def _sc_agg(h0, h1, src3, dst3, zrows, n_pad):
    nhalf = dst3.shape[2]
    nchunk = 2 * nhalf
    ept = nchunk * K
    rpt = zrows.shape[0]
    half = h0.shape[1]
    mesh = plsc.VectorSubcoreMesh(
        core_axis_name="c", subcore_axis_name="s", num_cores=NC, num_subcores=NS
    )

    @functools.partial(
        pl.kernel,
        out_type=jax.ShapeDtypeStruct((NC, n_pad, half), jnp.float32),
        mesh=mesh,
        scratch_types=[
            pltpu.VMEM((ept,), jnp.int32),
            pltpu.VMEM((nhalf, K), jnp.int32),
            pltpu.VMEM((2, K, half), jnp.float32),
            pltpu.VMEM_SHARED((n_pad, half), jnp.float32),
            pltpu.SemaphoreType.DMA,
            pltpu.SemaphoreType.DMA,
            pltpu.SemaphoreType.DMA,
            pltpu.SemaphoreType.DMA,
        ],
    )
    def agg(h0_hbm, h1_hbm, src_hbm, dst_hbm, z_hbm, out_hbm,
            sidx, didx, rows, acc, gsem0, gsem1, ssem0, ssem1):
        cid = lax.axis_index("c")
        sid = lax.axis_index("s")
        pltpu.sync_copy(src_hbm.at[pl.ds(sid * ept, ept)], sidx)
        pltpu.sync_copy(z_hbm, acc.at[pl.ds(sid * rpt, rpt)])
        plsc.subcore_barrier()

        gsems = (gsem0, gsem1)
        ssems = (ssem0, ssem1)

        def run(tab):
            def drain_scatter(nb):
                pltpu.make_async_copy(
                    rows.at[nb], acc.at[didx.at[0]], ssems[nb]
                ).wait()

            pltpu.async_copy(tab.at[sidx.at[pl.ds(0, K)]], rows.at[0], gsem0)

            for hidx in range(2):
                if hidx:
                    drain_scatter(1)
                pltpu.sync_copy(dst_hbm.at[sid, hidx], didx)

                def chunk_pair(p, _):
                    for b in range(2):
                        nb = 1 - b
                        lch = p * 2 + b
                        ch = hidx * nhalf + lch
                        pltpu.make_async_copy(
                            tab.at[sidx.at[pl.ds(ch * K, K)]], rows.at[b],
                            gsems[b]
                        ).wait()
                        if b == 1:
                            drain_scatter(nb)
                        else:
                            @pl.when(p >= 1)
                            def _():
                                drain_scatter(nb)
                        @pl.when(ch + 1 < nchunk)
                        def _():
                            pltpu.async_copy(
                                tab.at[sidx.at[pl.ds((ch + 1) * K, K)]],
                                rows.at[nb], gsems[nb]
                            )
                        pltpu.async_copy(rows.at[b], acc.at[didx.at[lch]],
                                         ssems[b], add=True)
                    return _

                lax.fori_loop(0, nhalf // 2, chunk_pair, None)

            drain_scatter(1)

        @pl.when(cid == 0)
        def _():
            run(h0_hbm)

        @pl.when(cid == 1)
        def _():
            run(h1_hbm)

        plsc.subcore_barrier()
        pltpu.sync_copy(
            acc.at[pl.ds(sid * rpt, rpt)],
            out_hbm.at[cid, pl.ds(sid * rpt, rpt)],
        )

    return agg(h0, h1, src3, dst3, zrows)


def _tc_conv(h0, h1, agg, w1, b1, w2, b2, scale, shift, epsv, rowb):
    n, half = h0.shape
    d = 2 * half
    grid = (n // rowb,)

    def body(eps_ref, h0_ref, h1_ref, a_ref, w1_ref, b1_ref, w2_ref, b2_ref,
             sc_ref, sh_ref, o0_ref, o1_ref):
        h = jnp.concatenate([h0_ref[...], h1_ref[...]], axis=1)
        a = jnp.concatenate([a_ref[0], a_ref[1]], axis=1)
        z = eps_ref[0, 0] * h + a
        z = jnp.maximum(
            jnp.dot(z, w1_ref[...], preferred_element_type=jnp.float32)
            + b1_ref[...], 0.0)
        z = jnp.maximum(
            jnp.dot(z, w2_ref[...], preferred_element_type=jnp.float32)
            + b2_ref[...], 0.0)
        z = z * sc_ref[...] + sh_ref[...]
        o0_ref[...] = z[:, :half]
        o1_ref[...] = z[:, half:]

    return pl.pallas_call(
        body,
        grid=grid,
        in_specs=[
            pl.BlockSpec(memory_space=pltpu.SMEM),
            pl.BlockSpec((rowb, half), lambda i: (i, 0)),
            pl.BlockSpec((rowb, half), lambda i: (i, 0)),
            pl.BlockSpec((NC, rowb, half), lambda i: (0, i, 0)),
            pl.BlockSpec((d, d), lambda i: (0, 0)),
            pl.BlockSpec((1, d), lambda i: (0, 0)),
            pl.BlockSpec((d, d), lambda i: (0, 0)),
            pl.BlockSpec((1, d), lambda i: (0, 0)),
            pl.BlockSpec((1, d), lambda i: (0, 0)),
            pl.BlockSpec((1, d), lambda i: (0, 0)),
        ],
        out_specs=[
            pl.BlockSpec((rowb, half), lambda i: (i, 0)),
            pl.BlockSpec((rowb, half), lambda i: (i, 0)),
        ],
        out_shape=[jax.ShapeDtypeStruct((n, half), jnp.float32)] * 2,
    )(epsv, h0, h1, agg, w1, b1, w2, b2, scale, shift)


def _tc_mlp(h0, h1, wbs, rowb):
    n, half = h0.shape
    d = 2 * half
    nout = wbs[-1][0].shape[1]
    grid = (n // rowb,)
    nl = len(wbs)

    def body(h0_ref, h1_ref, *refs):
        o_ref = refs[-1]
        z = jnp.concatenate([h0_ref[...], h1_ref[...]], axis=1)
        for i in range(nl):
            w_ref, b_ref = refs[2 * i], refs[2 * i + 1]
            z = jnp.dot(z, w_ref[...], preferred_element_type=jnp.float32) \
                + b_ref[...]
            if i < nl - 1:
                z = jnp.maximum(z, 0.0)
        o_ref[...] = z

    in_specs = [
        pl.BlockSpec((rowb, half), lambda i: (i, 0)),
        pl.BlockSpec((rowb, half), lambda i: (i, 0)),
    ]
    args = [h0, h1]
    for w, b in wbs:
        in_specs.append(pl.BlockSpec(w.shape, lambda i: (0, 0)))
        in_specs.append(pl.BlockSpec((1, b.shape[1]), lambda i: (0, 0)))
        args.append(w)
        args.append(b)

    return pl.pallas_call(
        body,
        grid=grid,
        in_specs=in_specs,
        out_specs=pl.BlockSpec((rowb, nout), lambda i: (i, 0)),
        out_shape=jax.ShapeDtypeStruct((n, nout), jnp.float32),
    )(*args)


def kernel(x, edge_index, params):
    n, d = x.shape
    e = edge_index.shape[1]
    half = d // 2

    ept = _round_up(-(-e // NS), 4 * K)
    ep = ept * NS
    nchunk = ept // K
    rpt = _round_up(-(-(n + 1) // NS), 8)
    n_pad = rpt * NS
    padn = ep - e
    pad_src = (jnp.arange(padn, dtype=jnp.int32) * 37) % n
    pad_dst = n + (jnp.arange(padn, dtype=jnp.int32) % (n_pad - n))
    src3 = jnp.concatenate([edge_index[0], pad_src])
    dst3 = jnp.concatenate([edge_index[1], pad_dst]).reshape(
        NS, 2, nchunk // 2, K)
    zrows = jnp.zeros((rpt, half), jnp.float32)

    rowb = 400 if n % 400 == 0 else 8

    h0 = x[:, :half]
    h1 = x[:, half:]
    for p in params["convs"]:
        agg = _sc_agg(h0, h1, src3, dst3, zrows, n_pad)
        epsv = jnp.reshape(1.0 + p["eps"], (1, 1)).astype(jnp.float32)
        scale = (p["gamma"] / jnp.sqrt(1.0 + 1e-5)).reshape(1, d)
        shift = p["beta"].reshape(1, d)
        h0, h1 = _tc_conv(h0, h1, agg, p["W1"], p["b1"].reshape(1, -1),
                          p["W2"], p["b2"].reshape(1, -1), scale, shift,
                          epsv, rowb)

    wbs = [(lp["W"], lp["b"].reshape(1, -1)) for lp in params["mlp"]]
    return _tc_mlp(h0, h1, wbs, rowb)


# speedup vs baseline: 1.0038x; 1.0038x over previous
import functools

import jax
import jax.numpy as jnp
from jax import lax
from jax.experimental import pallas as pl
from jax.experimental.pallas import tpu as pltpu
from jax.experimental.pallas import tpu_sc as plsc

NC = 2
NS = 16
K = 128


def _round_up(a, b):
    return -(-a // b) * b


def _sc_agg(h0, h1, src3, dst3, zrows, n_pad):
    nhalf = dst3.shape[2]
    nchunk = 2 * nhalf
    ept = nchunk * K
    rpt = zrows.shape[0]
    half = h0.shape[1]
    mesh = plsc.VectorSubcoreMesh(
        core_axis_name="c", subcore_axis_name="s", num_cores=NC, num_subcores=NS
    )

    @functools.partial(
        pl.kernel,
        out_type=jax.ShapeDtypeStruct((NC, n_pad, half), jnp.float32),
        mesh=mesh,
        scratch_types=[
            pltpu.VMEM((ept,), jnp.int32),
            pltpu.VMEM((nhalf, K), jnp.int32),
            pltpu.VMEM((2, K, half), jnp.float32),
            pltpu.VMEM_SHARED((n_pad, half), jnp.float32),
            pltpu.SemaphoreType.DMA,
            pltpu.SemaphoreType.DMA,
            pltpu.SemaphoreType.DMA,
            pltpu.SemaphoreType.DMA,
        ],
    )
    def agg(h0_hbm, h1_hbm, src_hbm, dst_hbm, z_hbm, out_hbm,
            sidx, didx, rows, acc, sem0, sem1, ssem0, ssem1):
        cid = lax.axis_index("c")
        sid = lax.axis_index("s")
        pltpu.sync_copy(src_hbm.at[pl.ds(sid * ept, ept)], sidx)
        pltpu.sync_copy(z_hbm, acc.at[pl.ds(sid * rpt, rpt)])
        plsc.subcore_barrier()

        def run(tab):
            sems = (sem0, sem1)
            ssems = (ssem0, ssem1)
            pltpu.async_copy(tab.at[sidx.at[pl.ds(0, K)]], rows.at[0], sem0)

            for hidx in range(2):
                pltpu.sync_copy(dst_hbm.at[sid, hidx], didx)

                def chunk_pair(p, _):
                    for b in range(2):
                        lch = p * 2 + b
                        ch = hidx * nhalf + lch
                        pltpu.make_async_copy(
                            tab.at[sidx.at[pl.ds(ch * K, K)]], rows.at[b],
                            sems[b]
                        ).wait()
                        pltpu.async_copy(rows.at[b], acc.at[didx.at[lch]],
                                         ssems[b], add=True)

                        @pl.when(lch > 0)
                        def _():
                            pltpu.make_async_copy(
                                rows.at[1 - b], acc.at[didx.at[lch - 1]],
                                ssems[1 - b]
                            ).wait()

                        @pl.when(ch + 1 < nchunk)
                        def _():
                            pltpu.async_copy(
                                tab.at[sidx.at[pl.ds((ch + 1) * K, K)]],
                                rows.at[1 - b], sems[1 - b]
                            )
                    return _

                lax.fori_loop(0, nhalf // 2, chunk_pair, None)
                pltpu.make_async_copy(
                    rows.at[1], acc.at[didx.at[nhalf - 1]], ssems[1]
                ).wait()

        @pl.when(cid == 0)
        def _():
            run(h0_hbm)

        @pl.when(cid == 1)
        def _():
            run(h1_hbm)

        plsc.subcore_barrier()
        pltpu.sync_copy(
            acc.at[pl.ds(sid * rpt, rpt)],
            out_hbm.at[cid, pl.ds(sid * rpt, rpt)],
        )

    return agg(h0, h1, src3, dst3, zrows)


def _tc_conv(h0, h1, agg, w1, b1, w2, b2, scale, shift, epsv, rowb):
    n, half = h0.shape
    d = 2 * half
    grid = (n // rowb,)

    def body(eps_ref, h0_ref, h1_ref, a_ref, w1_ref, b1_ref, w2_ref, b2_ref,
             sc_ref, sh_ref, o0_ref, o1_ref):
        h = jnp.concatenate([h0_ref[...], h1_ref[...]], axis=1)
        a = jnp.concatenate([a_ref[0], a_ref[1]], axis=1)
        z = eps_ref[0, 0] * h + a
        z = jnp.maximum(
            jnp.dot(z, w1_ref[...], preferred_element_type=jnp.float32)
            + b1_ref[...], 0.0)
        z = jnp.maximum(
            jnp.dot(z, w2_ref[...], preferred_element_type=jnp.float32)
            + b2_ref[...], 0.0)
        z = z * sc_ref[...] + sh_ref[...]
        o0_ref[...] = z[:, :half]
        o1_ref[...] = z[:, half:]

    return pl.pallas_call(
        body,
        grid=grid,
        in_specs=[
            pl.BlockSpec(memory_space=pltpu.SMEM),
            pl.BlockSpec((rowb, half), lambda i: (i, 0)),
            pl.BlockSpec((rowb, half), lambda i: (i, 0)),
            pl.BlockSpec((NC, rowb, half), lambda i: (0, i, 0)),
            pl.BlockSpec((d, d), lambda i: (0, 0)),
            pl.BlockSpec((1, d), lambda i: (0, 0)),
            pl.BlockSpec((d, d), lambda i: (0, 0)),
            pl.BlockSpec((1, d), lambda i: (0, 0)),
            pl.BlockSpec((1, d), lambda i: (0, 0)),
            pl.BlockSpec((1, d), lambda i: (0, 0)),
        ],
        out_specs=[
            pl.BlockSpec((rowb, half), lambda i: (i, 0)),
            pl.BlockSpec((rowb, half), lambda i: (i, 0)),
        ],
        out_shape=[jax.ShapeDtypeStruct((n, half), jnp.float32)] * 2,
    )(epsv, h0, h1, agg, w1, b1, w2, b2, scale, shift)


def _tc_mlp(h0, h1, wbs, rowb):
    n, half = h0.shape
    d = 2 * half
    nout = wbs[-1][0].shape[1]
    grid = (n // rowb,)
    nl = len(wbs)

    def body(h0_ref, h1_ref, *refs):
        o_ref = refs[-1]
        z = jnp.concatenate([h0_ref[...], h1_ref[...]], axis=1)
        for i in range(nl):
            w_ref, b_ref = refs[2 * i], refs[2 * i + 1]
            z = jnp.dot(z, w_ref[...], preferred_element_type=jnp.float32) \
                + b_ref[...]
            if i < nl - 1:
                z = jnp.maximum(z, 0.0)
        o_ref[...] = z

    in_specs = [
        pl.BlockSpec((rowb, half), lambda i: (i, 0)),
        pl.BlockSpec((rowb, half), lambda i: (i, 0)),
    ]
    args = [h0, h1]
    for w, b in wbs:
        in_specs.append(pl.BlockSpec(w.shape, lambda i: (0, 0)))
        in_specs.append(pl.BlockSpec((1, b.shape[1]), lambda i: (0, 0)))
        args.append(w)
        args.append(b)

    return pl.pallas_call(
        body,
        grid=grid,
        in_specs=in_specs,
        out_specs=pl.BlockSpec((rowb, nout), lambda i: (i, 0)),
        out_shape=jax.ShapeDtypeStruct((n, nout), jnp.float32),
    )(*args)


def kernel(x, edge_index, params):
    n, d = x.shape
    e = edge_index.shape[1]
    half = d // 2

    ept = _round_up(-(-e // NS), 4 * K)
    ep = ept * NS
    nchunk = ept // K
    rpt = _round_up(-(-(n + 1) // NS), 8)
    n_pad = rpt * NS
    padn = ep - e
    pad_src = (jnp.arange(padn, dtype=jnp.int32) * 37) % n
    pad_dst = n + (jnp.arange(padn, dtype=jnp.int32) % (n_pad - n))
    src3 = jnp.concatenate([edge_index[0], pad_src])
    dst3 = jnp.concatenate([edge_index[1], pad_dst]).reshape(
        NS, 2, nchunk // 2, K)
    zrows = jnp.zeros((rpt, half), jnp.float32)

    rowb = 400 if n % 400 == 0 else 8

    h0 = x[:, :half]
    h1 = x[:, half:]
    for p in params["convs"]:
        agg = _sc_agg(h0, h1, src3, dst3, zrows, n_pad)
        epsv = jnp.reshape(1.0 + p["eps"], (1, 1)).astype(jnp.float32)
        scale = (p["gamma"] / jnp.sqrt(1.0 + 1e-5)).reshape(1, d)
        shift = p["beta"].reshape(1, d)
        h0, h1 = _tc_conv(h0, h1, agg, p["W1"], p["b1"].reshape(1, -1),
                          p["W2"], p["b2"].reshape(1, -1), scale, shift,
                          epsv, rowb)

    wbs = [(lp["W"], lp["b"].reshape(1, -1)) for lp in params["mlp"]]
    return _tc_mlp(h0, h1, wbs, rowb)


# speedup vs baseline: 1.1467x; 1.1423x over previous
import functools

import jax
import jax.numpy as jnp
from jax import lax
from jax.experimental import pallas as pl
from jax.experimental.pallas import tpu as pltpu
from jax.experimental.pallas import tpu_sc as plsc

NC = 2
NS = 16
K = 128


def _round_up(a, b):
    return -(-a // b) * b


def _sc_agg(h0, h1, src3, dst3, zrows, n_pad):
    nhalf = dst3.shape[2]
    nchunk = 2 * nhalf
    ept = nchunk * K
    rpt = zrows.shape[0]
    half = h0.shape[1]
    mesh = plsc.VectorSubcoreMesh(
        core_axis_name="c", subcore_axis_name="s", num_cores=NC, num_subcores=NS
    )

    @functools.partial(
        pl.kernel,
        out_type=jax.ShapeDtypeStruct((NC, n_pad, half), jnp.float32),
        mesh=mesh,
        scratch_types=[
            pltpu.VMEM((ept,), jnp.int32),
            pltpu.VMEM((nhalf, K), jnp.int32),
            pltpu.VMEM((2, K, half), jnp.float32),
            pltpu.VMEM_SHARED((n_pad, half), jnp.float32),
            pltpu.SemaphoreType.DMA,
            pltpu.SemaphoreType.DMA,
        ],
    )
    def agg(h0_hbm, h1_hbm, src_hbm, dst_hbm, z_hbm, out_hbm,
            sidx, didx, rows, acc, sem0, sem1):
        cid = lax.axis_index("c")
        sid = lax.axis_index("s")
        pltpu.sync_copy(src_hbm.at[pl.ds(sid * ept, ept)], sidx)
        pltpu.sync_copy(z_hbm, acc.at[pl.ds(sid * rpt, rpt)])
        plsc.subcore_barrier()

        def run(tab):
            pltpu.async_copy(tab.at[sidx.at[pl.ds(0, K)]], rows.at[0], sem0)
            pltpu.async_copy(tab.at[sidx.at[pl.ds(K, K)]], rows.at[1], sem1)
            sems = (sem0, sem1)

            for hidx in range(2):
                pltpu.sync_copy(dst_hbm.at[sid, hidx], didx)

                def chunk_pair(p, _):
                    for b in range(2):
                        lch = p * 2 + b
                        ch = hidx * nhalf + lch
                        pltpu.make_async_copy(
                            tab.at[sidx.at[pl.ds(ch * K, K)]], rows.at[b],
                            sems[b]
                        ).wait()
                        pltpu.sync_copy(rows.at[b], acc.at[didx.at[lch]],
                                        add=True)

                        @pl.when(ch + 2 < nchunk)
                        def _():
                            pltpu.async_copy(
                                tab.at[sidx.at[pl.ds((ch + 2) * K, K)]],
                                rows.at[b], sems[b]
                            )
                    return _

                lax.fori_loop(0, nhalf // 2, chunk_pair, None)

        @pl.when(cid == 0)
        def _():
            run(h0_hbm)

        @pl.when(cid == 1)
        def _():
            run(h1_hbm)

        plsc.subcore_barrier()
        pltpu.sync_copy(
            acc.at[pl.ds(sid * rpt, rpt)],
            out_hbm.at[cid, pl.ds(sid * rpt, rpt)],
        )

    return agg(h0, h1, src3, dst3, zrows)


def _tc_conv(h0, h1, agg, w1, b1, w2, b2, scale, shift, epsv, rowb):
    n, half = h0.shape
    d = 2 * half
    grid = (n // rowb,)

    def body(eps_ref, h0_ref, h1_ref, a_ref, w1_ref, b1_ref, w2_ref, b2_ref,
             sc_ref, sh_ref, o0_ref, o1_ref):
        h = jnp.concatenate([h0_ref[...], h1_ref[...]], axis=1)
        a = jnp.concatenate([a_ref[0], a_ref[1]], axis=1)
        z = eps_ref[0, 0] * h + a
        z = jnp.maximum(
            jnp.dot(z, w1_ref[...], preferred_element_type=jnp.float32)
            + b1_ref[...], 0.0)
        z = jnp.maximum(
            jnp.dot(z, w2_ref[...], preferred_element_type=jnp.float32)
            + b2_ref[...], 0.0)
        z = z * sc_ref[...] + sh_ref[...]
        o0_ref[...] = z[:, :half]
        o1_ref[...] = z[:, half:]

    return pl.pallas_call(
        body,
        grid=grid,
        in_specs=[
            pl.BlockSpec(memory_space=pltpu.SMEM),
            pl.BlockSpec((rowb, half), lambda i: (i, 0)),
            pl.BlockSpec((rowb, half), lambda i: (i, 0)),
            pl.BlockSpec((NC, rowb, half), lambda i: (0, i, 0)),
            pl.BlockSpec((d, d), lambda i: (0, 0)),
            pl.BlockSpec((1, d), lambda i: (0, 0)),
            pl.BlockSpec((d, d), lambda i: (0, 0)),
            pl.BlockSpec((1, d), lambda i: (0, 0)),
            pl.BlockSpec((1, d), lambda i: (0, 0)),
            pl.BlockSpec((1, d), lambda i: (0, 0)),
        ],
        out_specs=[
            pl.BlockSpec((rowb, half), lambda i: (i, 0)),
            pl.BlockSpec((rowb, half), lambda i: (i, 0)),
        ],
        out_shape=[jax.ShapeDtypeStruct((n, half), jnp.float32)] * 2,
    )(epsv, h0, h1, agg, w1, b1, w2, b2, scale, shift)


def _tc_mlp(h0, h1, wbs, rowb):
    n, half = h0.shape
    d = 2 * half
    nout = wbs[-1][0].shape[1]
    grid = (n // rowb,)
    nl = len(wbs)

    def body(h0_ref, h1_ref, *refs):
        o_ref = refs[-1]
        z = jnp.concatenate([h0_ref[...], h1_ref[...]], axis=1)
        for i in range(nl):
            w_ref, b_ref = refs[2 * i], refs[2 * i + 1]
            z = jnp.dot(z, w_ref[...], preferred_element_type=jnp.float32) \
                + b_ref[...]
            if i < nl - 1:
                z = jnp.maximum(z, 0.0)
        o_ref[...] = z

    in_specs = [
        pl.BlockSpec((rowb, half), lambda i: (i, 0)),
        pl.BlockSpec((rowb, half), lambda i: (i, 0)),
    ]
    args = [h0, h1]
    for w, b in wbs:
        in_specs.append(pl.BlockSpec(w.shape, lambda i: (0, 0)))
        in_specs.append(pl.BlockSpec((1, b.shape[1]), lambda i: (0, 0)))
        args.append(w)
        args.append(b)

    return pl.pallas_call(
        body,
        grid=grid,
        in_specs=in_specs,
        out_specs=pl.BlockSpec((rowb, nout), lambda i: (i, 0)),
        out_shape=jax.ShapeDtypeStruct((n, nout), jnp.float32),
    )(*args)


def kernel(x, edge_index, params):
    n, d = x.shape
    e = edge_index.shape[1]
    half = d // 2

    ept = _round_up(-(-e // NS), 4 * K)
    ep = ept * NS
    nchunk = ept // K
    rpt = _round_up(-(-(n + 1) // NS), 8)
    n_pad = rpt * NS
    padn = ep - e
    pad_src = (jnp.arange(padn, dtype=jnp.int32) * 37) % n
    pad_dst = n + (jnp.arange(padn, dtype=jnp.int32) % (n_pad - n))
    src3 = jnp.concatenate([edge_index[0], pad_src])
    dst3 = jnp.concatenate([edge_index[1], pad_dst]).reshape(
        NS, 2, nchunk // 2, K)
    zrows = jnp.zeros((rpt, half), jnp.float32)

    rowb = 400 if n % 400 == 0 else 8

    h0 = x[:, :half]
    h1 = x[:, half:]
    for p in params["convs"]:
        agg = _sc_agg(h0, h1, src3, dst3, zrows, n_pad)
        epsv = jnp.reshape(1.0 + p["eps"], (1, 1)).astype(jnp.float32)
        scale = (p["gamma"] / jnp.sqrt(1.0 + 1e-5)).reshape(1, d)
        shift = p["beta"].reshape(1, d)
        h0, h1 = _tc_conv(h0, h1, agg, p["W1"], p["b1"].reshape(1, -1),
                          p["W2"], p["b2"].reshape(1, -1), scale, shift,
                          epsv, rowb)

    wbs = [(lp["W"], lp["b"].reshape(1, -1)) for lp in params["mlp"]]
    return _tc_mlp(h0, h1, wbs, rowb)
